# baseline (speedup 1.0000x reference)
import functools

import jax
import jax.numpy as jnp
from jax import lax
from jax.experimental import pallas as pl
from jax.experimental.pallas import tpu as pltpu
from jax.experimental.pallas import tpu_sc as plsc

NUM_CLASSES = 1000000
EMBED_DIM = 64
BATCH = 16384

_INFO = plsc.get_sparse_core_info()
_NC = _INFO.num_cores
_NS = _INFO.num_subcores
_NW = _NC * _NS
_L = _INFO.num_lanes

_B_PER_W = BATCH // _NW
_CHUNK = 128
_NCHUNK = _B_PER_W // _CHUNK
_NPASS = 2
_CPP = _NCHUNK // _NPASS
_PASS_TASKS = _B_PER_W // _NPASS
_GROUPS = _PASS_TASKS // _L


def _sc_kernel(xrow_hbm, yrow_hbm, xraw_hbm, yraw_hbm, tab_hbm, out_hbm,
               xrow_v, yrow_v, xraw_v, yraw_v, xrows_v, yrows_v, out_v, sem):
    wid = lax.axis_index("s") * _NC + lax.axis_index("c")

    pltpu.sync_copy(xrow_hbm.at[wid], xrow_v)
    pltpu.sync_copy(yrow_hbm.at[wid], yrow_v)
    pltpu.sync_copy(xraw_hbm.at[wid], xraw_v)
    pltpu.sync_copy(yraw_hbm.at[wid], yraw_v)

    iota = lax.broadcasted_iota(jnp.int32, (_L,), 0)
    ones = jnp.ones((_L,), jnp.int32)

    for p in range(_NPASS):
        copies = []
        for cc in range(_CPP):
            c = p * _CPP + cc
            copies.append(pltpu.async_copy(
                tab_hbm.at[xrow_v.at[c]],
                xrows_v.at[pl.ds(cc * _CHUNK, _CHUNK)], sem))
            copies.append(pltpu.async_copy(
                tab_hbm.at[yrow_v.at[c]],
                yrows_v.at[pl.ds(cc * _CHUNK, _CHUNK)], sem))
        for cp in copies:
            cp.wait()

        def group_body(g, carry):
            rowv = g * _L + iota
            c = p * _CPP + g // (_CHUNK // _L)
            off = (g % (_CHUNK // _L)) * _L
            xpar = (xraw_v[c, pl.ds(off, _L)] & ones) * EMBED_DIM
            ypar = (yraw_v[c, pl.ds(off, _L)] & ones) * EMBED_DIM
            acc = jnp.zeros((_L,), jnp.float32)
            for j in range(EMBED_DIM):
                gx = plsc.load_gather(xrows_v, [rowv, xpar + j])
                gy = plsc.load_gather(yrows_v, [rowv, ypar + j])
                acc = acc + gx * gy
            out_v[pl.ds(p * _PASS_TASKS + g * _L, _L)] = (
                1.0 / (1.0 + jnp.exp(acc)))
            return carry

        lax.fori_loop(0, _GROUPS, group_body, 0)

    pltpu.sync_copy(out_v, out_hbm.at[pl.ds(wid * _B_PER_W, _B_PER_W)])


@jax.jit
def kernel(table, x_idx, y_idx):
    tab2 = table.reshape(NUM_CLASSES // 2, 2 * EMBED_DIM)
    xrow = (x_idx >> 1).reshape(_NW, _NCHUNK, _CHUNK)
    yrow = (y_idx >> 1).reshape(_NW, _NCHUNK, _CHUNK)
    xraw = x_idx.reshape(_NW, _NCHUNK, _CHUNK)
    yraw = y_idx.reshape(_NW, _NCHUNK, _CHUNK)
    mesh = plsc.VectorSubcoreMesh(core_axis_name="c", subcore_axis_name="s")
    run = functools.partial(
        pl.kernel, mesh=mesh,
        compiler_params=pltpu.CompilerParams(needs_layout_passes=False),
        out_type=jax.ShapeDtypeStruct((BATCH,), jnp.float32),
        scratch_types=[
            pltpu.VMEM((_NCHUNK, _CHUNK), jnp.int32),
            pltpu.VMEM((_NCHUNK, _CHUNK), jnp.int32),
            pltpu.VMEM((_NCHUNK, _CHUNK), jnp.int32),
            pltpu.VMEM((_NCHUNK, _CHUNK), jnp.int32),
            pltpu.VMEM((_PASS_TASKS, 2 * EMBED_DIM), jnp.float32),
            pltpu.VMEM((_PASS_TASKS, 2 * EMBED_DIM), jnp.float32),
            pltpu.VMEM((_B_PER_W,), jnp.float32),
            pltpu.SemaphoreType.DMA,
        ],
    )(_sc_kernel)
    return run(xrow, yrow, xraw, yraw, tab2)


# speedup vs baseline: 2.0540x; 2.0540x over previous
import functools

import jax
import jax.numpy as jnp
from jax import lax
from jax.experimental import pallas as pl
from jax.experimental.pallas import tpu as pltpu
from jax.experimental.pallas import tpu_sc as plsc

NUM_CLASSES = 1000000
EMBED_DIM = 64
BATCH = 16384
_TILE = 8

_INFO = plsc.get_sparse_core_info()
_NC = _INFO.num_cores
_NS = _INFO.num_subcores
_NW = _NC * _NS
_L = _INFO.num_lanes

_B_PER_W = BATCH // _NW
_CHUNK = 32
_NCHUNK = _B_PER_W // _CHUNK
_IROWS = _B_PER_W // 128
_GROUPS = _CHUNK // _L


def _sc_kernel(xslab_hbm, yslab_hbm, xsub_hbm, ysub_hbm, tab_hbm, out_hbm,
               xslabidx_v, yslabidx_v, xsub_v, ysub_v,
               xslabs_v, yslabs_v, out_v, sem):
    wid = lax.axis_index("s") * _NC + lax.axis_index("c")

    pltpu.sync_copy(xslab_hbm.at[wid], xslabidx_v)
    pltpu.sync_copy(yslab_hbm.at[wid], yslabidx_v)
    pltpu.sync_copy(xsub_hbm.at[wid], xsub_v)
    pltpu.sync_copy(ysub_hbm.at[wid], ysub_v)

    iota = lax.broadcasted_iota(jnp.int32, (_L,), 0)

    def chunk_body(c, carry):
        for g in range(_GROUPS):
            base = c * _CHUNK + g * _L
            row = base // 128
            lane = base % 128
            xvec = xslabidx_v[row, pl.ds(lane, _L)]
            yvec = yslabidx_v[row, pl.ds(lane, _L)]
            for k in range(_L):
                t = g * _L + k
                pltpu.async_copy(tab_hbm.at[xvec[k]], xslabs_v.at[t], sem)
                pltpu.async_copy(tab_hbm.at[yvec[k]], yslabs_v.at[t], sem)
        for _ in range(2 * _CHUNK):
            pltpu.make_async_copy(tab_hbm.at[0], xslabs_v.at[0], sem).wait()

        def group(g, carry2):
            taskv = g * _L + iota
            base = c * _CHUNK + g * _L
            row = base // 128
            lane = base % 128
            xsub = xsub_v[row, pl.ds(lane, _L)]
            ysub = ysub_v[row, pl.ds(lane, _L)]
            acc = jnp.zeros((_L,), jnp.float32)
            for j in range(EMBED_DIM):
                colv = jnp.full((_L,), j, jnp.int32)
                gx = plsc.load_gather(xslabs_v, [taskv, xsub, colv])
                gy = plsc.load_gather(yslabs_v, [taskv, ysub, colv])
                acc = acc + gx * gy
            out_v[pl.ds(c * _CHUNK + g * _L, _L)] = 1.0 / (1.0 + jnp.exp(acc))
            return carry2

        lax.fori_loop(0, _GROUPS, group, 0)
        return carry

    lax.fori_loop(0, _NCHUNK, chunk_body, 0)

    pltpu.sync_copy(out_v, out_hbm.at[pl.ds(wid * _B_PER_W, _B_PER_W)])


@jax.jit
def kernel(table, x_idx, y_idx):
    tab3 = table.reshape(NUM_CLASSES // _TILE, _TILE, EMBED_DIM)
    xslab = (x_idx >> 3).reshape(_NW, _IROWS, 128)
    yslab = (y_idx >> 3).reshape(_NW, _IROWS, 128)
    xsub = (x_idx & 7).reshape(_NW, _IROWS, 128)
    ysub = (y_idx & 7).reshape(_NW, _IROWS, 128)
    mesh = plsc.VectorSubcoreMesh(core_axis_name="c", subcore_axis_name="s")
    run = functools.partial(
        pl.kernel, mesh=mesh,
        compiler_params=pltpu.CompilerParams(needs_layout_passes=False),
        out_type=jax.ShapeDtypeStruct((BATCH,), jnp.float32),
        scratch_types=[
            pltpu.VMEM((_IROWS, 128), jnp.int32),
            pltpu.VMEM((_IROWS, 128), jnp.int32),
            pltpu.VMEM((_IROWS, 128), jnp.int32),
            pltpu.VMEM((_IROWS, 128), jnp.int32),
            pltpu.VMEM((_CHUNK, _TILE, EMBED_DIM), jnp.float32),
            pltpu.VMEM((_CHUNK, _TILE, EMBED_DIM), jnp.float32),
            pltpu.VMEM((_B_PER_W,), jnp.float32),
            pltpu.SemaphoreType.DMA,
        ],
    )(_sc_kernel)
    return run(xslab, yslab, xsub, ysub, tab3)


# speedup vs baseline: 2.2641x; 1.1023x over previous
import functools

import jax
import jax.numpy as jnp
from jax import lax
from jax.experimental import pallas as pl
from jax.experimental.pallas import tpu as pltpu
from jax.experimental.pallas import tpu_sc as plsc

NUM_CLASSES = 1000000
EMBED_DIM = 64
BATCH = 16384
_TILE = 8

_INFO = plsc.get_sparse_core_info()
_NC = _INFO.num_cores
_NS = _INFO.num_subcores
_NW = _NC * _NS
_L = _INFO.num_lanes

_B_PER_W = BATCH // _NW
_CHUNK = _L
_NCHUNK = _B_PER_W // _CHUNK
_IROWS = _B_PER_W // 128


def _sc_kernel(xslab_hbm, yslab_hbm, xsub_hbm, ysub_hbm, tab_hbm, out_hbm,
               xslabidx_v, yslabidx_v, xsub_v, ysub_v,
               xslabs, yslabs, out_v, sems):
    wid = lax.axis_index("s") * _NC + lax.axis_index("c")

    pltpu.sync_copy(xslab_hbm.at[wid], xslabidx_v)
    pltpu.sync_copy(yslab_hbm.at[wid], yslabidx_v)
    pltpu.sync_copy(xsub_hbm.at[wid], xsub_v)
    pltpu.sync_copy(ysub_hbm.at[wid], ysub_v)

    iota = lax.broadcasted_iota(jnp.int32, (_L,), 0)

    def fire(c, buf):
        base = c * _CHUNK
        row = base // 128
        lane = base % 128
        xvec = xslabidx_v[row, pl.ds(lane, _L)]
        yvec = yslabidx_v[row, pl.ds(lane, _L)]
        for k in range(_L):
            pltpu.async_copy(tab_hbm.at[xvec[k]], xslabs[buf].at[k],
                             sems[buf])
            pltpu.async_copy(tab_hbm.at[yvec[k]], yslabs[buf].at[k],
                             sems[buf])

    def drain(buf):
        for _ in range(2 * _CHUNK):
            pltpu.make_async_copy(tab_hbm.at[0], xslabs[buf].at[0],
                                  sems[buf]).wait()

    def compute(c, buf):
        base = c * _CHUNK
        row = base // 128
        lane = base % 128
        xsub = xsub_v[row, pl.ds(lane, _L)]
        ysub = ysub_v[row, pl.ds(lane, _L)]
        acc = jnp.zeros((_L,), jnp.float32)
        for j in range(EMBED_DIM):
            colv = jnp.full((_L,), j, jnp.int32)
            gx = plsc.load_gather(xslabs[buf], [iota, xsub, colv])
            gy = plsc.load_gather(yslabs[buf], [iota, ysub, colv])
            acc = acc + gx * gy
        out_v[pl.ds(base, _L)] = 1.0 / (1.0 + jnp.exp(acc))

    fire(0, 0)

    def pair_body(i, carry):
        c0 = 2 * i
        fire(c0 + 1, 1)
        drain(0)
        compute(c0, 0)

        @pl.when(i + 1 < _NCHUNK // 2)
        def _():
            fire(c0 + 2, 0)

        drain(1)
        compute(c0 + 1, 1)
        return carry

    lax.fori_loop(0, _NCHUNK // 2, pair_body, 0)

    pltpu.sync_copy(out_v, out_hbm.at[pl.ds(wid * _B_PER_W, _B_PER_W)])


@jax.jit
def kernel(table, x_idx, y_idx):
    tab3 = table.reshape(NUM_CLASSES // _TILE, _TILE, EMBED_DIM)
    xslab = (x_idx >> 3).reshape(_NW, _IROWS, 128)
    yslab = (y_idx >> 3).reshape(_NW, _IROWS, 128)
    xsub = (x_idx & 7).reshape(_NW, _IROWS, 128)
    ysub = (y_idx & 7).reshape(_NW, _IROWS, 128)
    mesh = plsc.VectorSubcoreMesh(core_axis_name="c", subcore_axis_name="s")
    run = functools.partial(
        pl.kernel, mesh=mesh,
        compiler_params=pltpu.CompilerParams(needs_layout_passes=False),
        out_type=jax.ShapeDtypeStruct((BATCH,), jnp.float32),
        scratch_types=[
            pltpu.VMEM((_IROWS, 128), jnp.int32),
            pltpu.VMEM((_IROWS, 128), jnp.int32),
            pltpu.VMEM((_IROWS, 128), jnp.int32),
            pltpu.VMEM((_IROWS, 128), jnp.int32),
            [pltpu.VMEM((_CHUNK, _TILE, EMBED_DIM), jnp.float32)
             for _ in range(2)],
            [pltpu.VMEM((_CHUNK, _TILE, EMBED_DIM), jnp.float32)
             for _ in range(2)],
            pltpu.VMEM((_B_PER_W,), jnp.float32),
            [pltpu.SemaphoreType.DMA for _ in range(2)],
        ],
    )(_sc_kernel)
    return run(xslab, yslab, xsub, ysub, tab3)


# speedup vs baseline: 2.3129x; 1.0216x over previous
import functools

import jax
import jax.numpy as jnp
from jax import lax
from jax.experimental import pallas as pl
from jax.experimental.pallas import tpu as pltpu
from jax.experimental.pallas import tpu_sc as plsc

NUM_CLASSES = 1000000
EMBED_DIM = 64
BATCH = 16384
_TILE = 8

_INFO = plsc.get_sparse_core_info()
_NC = _INFO.num_cores
_NS = _INFO.num_subcores
_NW = _NC * _NS
_L = _INFO.num_lanes

_B_PER_W = BATCH // _NW
_CHUNK = _L
_NCHUNK = _B_PER_W // _CHUNK
_IROWS = _B_PER_W // 128
_NBUF = 3


def _sc_kernel(xslab_hbm, yslab_hbm, xsub_hbm, ysub_hbm, tab_hbm, out_hbm,
               xslabidx_v, yslabidx_v, xsub_v, ysub_v,
               xslabs, yslabs, out_v, sems):
    wid = lax.axis_index("s") * _NC + lax.axis_index("c")

    pltpu.sync_copy(xslab_hbm.at[wid], xslabidx_v)
    pltpu.sync_copy(yslab_hbm.at[wid], yslabidx_v)
    pltpu.sync_copy(xsub_hbm.at[wid], xsub_v)
    pltpu.sync_copy(ysub_hbm.at[wid], ysub_v)

    iota = lax.broadcasted_iota(jnp.int32, (_L,), 0)

    def fire(c, buf):
        base = c * _CHUNK
        row = base // 128
        lane = base % 128
        xvec = xslabidx_v[row, pl.ds(lane, _L)]
        yvec = yslabidx_v[row, pl.ds(lane, _L)]
        for k in range(_L):
            pltpu.async_copy(tab_hbm.at[xvec[k]], xslabs[buf].at[k],
                             sems[buf])
            pltpu.async_copy(tab_hbm.at[yvec[k]], yslabs[buf].at[k],
                             sems[buf])

    def drain(buf):
        for _ in range(2 * _CHUNK):
            pltpu.make_async_copy(tab_hbm.at[0], xslabs[buf].at[0],
                                  sems[buf]).wait()

    def compute(c, buf):
        base = c * _CHUNK
        row = base // 128
        lane = base % 128
        xsub = xsub_v[row, pl.ds(lane, _L)]
        ysub = ysub_v[row, pl.ds(lane, _L)]
        acc = jnp.zeros((_L,), jnp.float32)
        for j in range(EMBED_DIM):
            colv = jnp.full((_L,), j, jnp.int32)
            gx = plsc.load_gather(xslabs[buf], [iota, xsub, colv])
            gy = plsc.load_gather(yslabs[buf], [iota, ysub, colv])
            acc = acc + gx * gy
        out_v[pl.ds(base, _L)] = 1.0 / (1.0 + jnp.exp(acc))

    for b in range(_NBUF):
        fire(b, b)

    def ring_body(i, carry):
        c0 = _NBUF * i
        for q in range(_NBUF):
            c = c0 + q

            @pl.when(c < _NCHUNK)
            def _():
                drain(q)
                compute(c, q)

            @pl.when(c + _NBUF < _NCHUNK)
            def _():
                fire(c + _NBUF, q)

        return carry

    lax.fori_loop(0, (_NCHUNK + _NBUF - 1) // _NBUF, ring_body, 0)

    pltpu.sync_copy(out_v, out_hbm.at[pl.ds(wid * _B_PER_W, _B_PER_W)])


@jax.jit
def kernel(table, x_idx, y_idx):
    tab3 = table.reshape(NUM_CLASSES // _TILE, _TILE, EMBED_DIM)
    xslab = (x_idx >> 3).reshape(_NW, _IROWS, 128)
    yslab = (y_idx >> 3).reshape(_NW, _IROWS, 128)
    xsub = (x_idx & 7).reshape(_NW, _IROWS, 128)
    ysub = (y_idx & 7).reshape(_NW, _IROWS, 128)
    mesh = plsc.VectorSubcoreMesh(core_axis_name="c", subcore_axis_name="s")
    run = functools.partial(
        pl.kernel, mesh=mesh,
        compiler_params=pltpu.CompilerParams(needs_layout_passes=False),
        out_type=jax.ShapeDtypeStruct((BATCH,), jnp.float32),
        scratch_types=[
            pltpu.VMEM((_IROWS, 128), jnp.int32),
            pltpu.VMEM((_IROWS, 128), jnp.int32),
            pltpu.VMEM((_IROWS, 128), jnp.int32),
            pltpu.VMEM((_IROWS, 128), jnp.int32),
            [pltpu.VMEM((_CHUNK, _TILE, EMBED_DIM), jnp.float32)
             for _ in range(_NBUF)],
            [pltpu.VMEM((_CHUNK, _TILE, EMBED_DIM), jnp.float32)
             for _ in range(_NBUF)],
            pltpu.VMEM((_B_PER_W,), jnp.float32),
            [pltpu.SemaphoreType.DMA for _ in range(_NBUF)],
        ],
    )(_sc_kernel)
    return run(xslab, yslab, xsub, ysub, tab3)


# speedup vs baseline: 2.3244x; 1.0050x over previous
import functools

import jax
import jax.numpy as jnp
from jax import lax
from jax.experimental import pallas as pl
from jax.experimental.pallas import tpu as pltpu
from jax.experimental.pallas import tpu_sc as plsc

NUM_CLASSES = 1000000
EMBED_DIM = 64
BATCH = 16384
_TILE = 8

_INFO = plsc.get_sparse_core_info()
_NC = _INFO.num_cores
_NS = _INFO.num_subcores
_NW = _NC * _NS
_L = _INFO.num_lanes

_B_PER_W = BATCH // _NW
_CHUNK = _L
_NCHUNK = _B_PER_W // _CHUNK
_IROWS = _B_PER_W // 128
_NBUF = 3


def _sc_kernel(xslab_hbm, yslab_hbm, xsub_hbm, ysub_hbm, tab_hbm, out_hbm,
               xslabidx_v, yslabidx_v, xsub_v, ysub_v,
               xslabs, yslabs, out_v, sems):
    wid = lax.axis_index("s") * _NC + lax.axis_index("c")

    staging = [
        pltpu.async_copy(xslab_hbm.at[wid], xslabidx_v, sems[0]),
        pltpu.async_copy(yslab_hbm.at[wid], yslabidx_v, sems[0]),
        pltpu.async_copy(xsub_hbm.at[wid], xsub_v, sems[0]),
        pltpu.async_copy(ysub_hbm.at[wid], ysub_v, sems[0]),
    ]
    for cp in staging:
        cp.wait()

    iota = lax.broadcasted_iota(jnp.int32, (_L,), 0)

    def fire(c, buf):
        base = c * _CHUNK
        row = base // 128
        lane = base % 128
        xvec = xslabidx_v[row, pl.ds(lane, _L)]
        yvec = yslabidx_v[row, pl.ds(lane, _L)]
        for k in range(_L):
            pltpu.async_copy(tab_hbm.at[xvec[k]], xslabs[buf].at[k],
                             sems[buf])
            pltpu.async_copy(tab_hbm.at[yvec[k]], yslabs[buf].at[k],
                             sems[buf])

    def drain(buf):
        for _ in range(2 * _CHUNK):
            pltpu.make_async_copy(tab_hbm.at[0], xslabs[buf].at[0],
                                  sems[buf]).wait()

    def compute(c, buf):
        base = c * _CHUNK
        row = base // 128
        lane = base % 128
        xsub = xsub_v[row, pl.ds(lane, _L)]
        ysub = ysub_v[row, pl.ds(lane, _L)]
        acc = jnp.zeros((_L,), jnp.float32)
        for j in range(EMBED_DIM):
            colv = jnp.full((_L,), j, jnp.int32)
            gx = plsc.load_gather(xslabs[buf], [iota, xsub, colv])
            gy = plsc.load_gather(yslabs[buf], [iota, ysub, colv])
            acc = acc + gx * gy
        out_v[pl.ds(base, _L)] = 1.0 / (1.0 + jnp.exp(acc))

    for b in range(_NBUF):
        fire(b, b)

    def ring_body(i, carry):
        c0 = _NBUF * i
        for q in range(_NBUF):
            c = c0 + q

            @pl.when(c < _NCHUNK)
            def _():
                drain(q)
                compute(c, q)

            @pl.when(c + _NBUF < _NCHUNK)
            def _():
                fire(c + _NBUF, q)

        return carry

    lax.fori_loop(0, (_NCHUNK + _NBUF - 1) // _NBUF, ring_body, 0)

    pltpu.sync_copy(out_v, out_hbm.at[pl.ds(wid * _B_PER_W, _B_PER_W)])


@jax.jit
def kernel(table, x_idx, y_idx):
    tab3 = table.reshape(NUM_CLASSES // _TILE, _TILE, EMBED_DIM)
    xslab = (x_idx >> 3).reshape(_NW, _IROWS, 128)
    yslab = (y_idx >> 3).reshape(_NW, _IROWS, 128)
    xsub = (x_idx & 7).reshape(_NW, _IROWS, 128)
    ysub = (y_idx & 7).reshape(_NW, _IROWS, 128)
    mesh = plsc.VectorSubcoreMesh(core_axis_name="c", subcore_axis_name="s")
    run = functools.partial(
        pl.kernel, mesh=mesh,
        compiler_params=pltpu.CompilerParams(needs_layout_passes=False),
        out_type=jax.ShapeDtypeStruct((BATCH,), jnp.float32),
        scratch_types=[
            pltpu.VMEM((_IROWS, 128), jnp.int32),
            pltpu.VMEM((_IROWS, 128), jnp.int32),
            pltpu.VMEM((_IROWS, 128), jnp.int32),
            pltpu.VMEM((_IROWS, 128), jnp.int32),
            [pltpu.VMEM((_CHUNK, _TILE, EMBED_DIM), jnp.float32)
             for _ in range(_NBUF)],
            [pltpu.VMEM((_CHUNK, _TILE, EMBED_DIM), jnp.float32)
             for _ in range(_NBUF)],
            pltpu.VMEM((_B_PER_W,), jnp.float32),
            [pltpu.SemaphoreType.DMA for _ in range(_NBUF)],
        ],
    )(_sc_kernel)
    return run(xslab, yslab, xsub, ysub, tab3)
